# baseline (speedup 1.0000x reference)
import functools

import jax
import jax.numpy as jnp
from jax import lax
from jax.experimental import pallas as pl
from jax.experimental.pallas import tpu as pltpu
from jax.experimental.pallas import tpu_sc as plsc

_EPS = 1e-12
_K = 128
_NBUF = 4
_ZB = 100


def _build_sc_gather(vocab, hid, tok, nc, ns):
    nw = nc * ns
    per_w = tok // nw
    nj = per_w // _K
    assert per_w % _K == 0 and nj >= _NBUF
    nfull = (nj // _NBUF) * _NBUF

    mesh = plsc.VectorSubcoreMesh(core_axis_name="c", subcore_axis_name="s")

    @functools.partial(
        pl.kernel,
        mesh=mesh,
        out_type=jax.ShapeDtypeStruct((tok, hid), jnp.float32),
        scratch_types=[
            pltpu.VMEM((nj, _K), jnp.int32),
            [pltpu.VMEM((_K, hid), jnp.float32) for _ in range(_NBUF)],
            [pltpu.SemaphoreType.DMA for _ in range(_NBUF)],
            [pltpu.SemaphoreType.DMA for _ in range(_NBUF)],
        ],
    )
    def sc_gather(table, idx, out, idx_v, bufs, gsems, osems):
        wid = lax.axis_index("s") * nc + lax.axis_index("c")
        base = wid * per_w
        pltpu.sync_copy(idx.at[wid], idx_v)

        def start_gather(j, b):
            pltpu.make_async_copy(table.at[idx_v.at[j]], bufs[b], gsems[b]).start()

        def wait_gather(b):
            pltpu.make_async_copy(table.at[idx_v.at[0]], bufs[b], gsems[b]).wait()

        def start_out(j, b):
            pltpu.make_async_copy(
                bufs[b], out.at[pl.ds(base + j * _K, _K)], osems[b]
            ).start()

        def wait_out(b):
            pltpu.make_async_copy(
                bufs[b], out.at[pl.ds(base, _K)], osems[b]
            ).wait()

        def half_step(j, b):
            ab = (b + _NBUF - 1) % _NBUF
            gb = (b + _NBUF - 2) % _NBUF
            wait_gather(b)
            start_out(j, b)

            @pl.when(jnp.logical_and(j >= 1, j + 2 < nj))
            def _():
                wait_out(ab)

            @pl.when(j + 2 < nj)
            def _():
                start_gather(j + 2, gb)

        for b in range(2):
            start_gather(b, b)

        def step(i, carry):
            for b in range(_NBUF):
                half_step(i * _NBUF + b, b)
            return carry

        lax.fori_loop(0, nfull // _NBUF, step, 0)
        for j in range(nfull, nj):
            half_step(j, j % _NBUF)
        for j in range(nj - 3, nj):
            wait_out(j % _NBUF)

    return sc_gather, nw, nj


def _ln_body(x_ref, sid_ref, pos_ref, dseg_ref, w_ref, b_ref, o_ref):
    sg = sid_ref[0]
    seg = lax.dot_general(
        sg, dseg_ref[...], (((0,), (0,)), ((), ())),
        preferred_element_type=jnp.float32,
    )
    e = x_ref[...] + pos_ref[...] + seg
    mu = jnp.mean(e, axis=-1, keepdims=True)
    d = e - mu
    var = jnp.mean(d * d, axis=-1, keepdims=True)
    inv = lax.rsqrt(var + _EPS)
    o_ref[...] = d * inv * w_ref[0][None, :] + b_ref[0][None, :]


def kernel(token_ids, segment_ids, word_emb, pos_emb, seg_emb, ln_w, ln_b):
    bsz, seq = token_ids.shape
    vocab, hid = word_emb.shape
    nseg = seg_emb.shape[0]
    assert nseg == 2
    tok = bsz * seq

    info = plsc.get_sparse_core_info()
    nc, ns = info.num_cores, info.num_subcores
    sc_gather, nw, nj = _build_sc_gather(vocab, hid, tok, nc, ns)

    idx = token_ids.astype(jnp.int32).reshape(nw, nj, _K)
    gathered = sc_gather(word_emb, idx)

    g = tok // _K
    tb = _ZB * _K
    assert g % _ZB == 0 and tb % seq == 0
    posseg0 = pos_emb[:seq] + seg_emb[0][None, :]
    pos2 = jnp.tile(posseg0, (tb // seq, 1))
    sid3 = segment_ids.astype(jnp.float32).reshape(tok // tb, 1, tb)
    dseg = (seg_emb[1] - seg_emb[0]).reshape(1, hid)

    out = pl.pallas_call(
        _ln_body,
        grid=(tok // tb,),
        in_specs=[
            pl.BlockSpec((tb, hid), lambda i: (i, 0)),
            pl.BlockSpec((1, 1, tb), lambda i: (i, 0, 0)),
            pl.BlockSpec((tb, hid), lambda i: (0, 0)),
            pl.BlockSpec((1, hid), lambda i: (0, 0)),
            pl.BlockSpec((1, hid), lambda i: (0, 0)),
            pl.BlockSpec((1, hid), lambda i: (0, 0)),
        ],
        out_specs=pl.BlockSpec((tb, hid), lambda i: (i, 0)),
        out_shape=jax.ShapeDtypeStruct((tok, hid), jnp.float32),
        compiler_params=pltpu.CompilerParams(dimension_semantics=("parallel",)),
    )(
        gathered,
        sid3,
        pos2,
        dseg,
        ln_w.reshape(1, hid),
        ln_b.reshape(1, hid),
    )
    return out.reshape(bsz, seq, hid)
